# baseline (speedup 1.0000x reference)
import functools

import jax
import jax.numpy as jnp
from jax import lax
from jax.experimental import pallas as pl
from jax.experimental.pallas import tpu as pltpu
from jax.experimental.pallas import tpu_sc as plsc

_B, _L = 4096, 200
_DW, _DP = 128, 16
_D = _DW + _DP
_NC, _NS = 2, 16
_NW = _NC * _NS
_TB = _B // _NW
_DT = _D // 8
_INNER = _NW * 8 * 128


def _build():
  mesh = plsc.VectorSubcoreMesh(core_axis_name="c", subcore_axis_name="s")

  @functools.partial(
      pl.kernel,
      mesh=mesh,
      compiler_params=pltpu.CompilerParams(needs_layout_passes=False),
      out_type=jax.ShapeDtypeStruct((_L, _DT, _NW * 8, 128), jnp.float32),
      scratch_types=[
          pltpu.VMEM((_TB, _L), jnp.int32),
          pltpu.VMEM((_TB, _L), jnp.int32),
          pltpu.VMEM((24, _DP), jnp.float32),
          pltpu.VMEM((2, _TB), jnp.int32),
          pltpu.VMEM((_TB, _DW), jnp.float32),
          pltpu.VMEM((2, _DT, 8, _TB), jnp.float32),
          pltpu.SemaphoreType.DMA,
          pltpu.SemaphoreType.DMA,
          pltpu.SemaphoreType.DMA,
      ],
  )
  def emb(x_hbm, p_hbm, ww_hbm, wp_hbm, out_hbm,
          xi, pi, wp_v, xt, wrows, cat, gsem, ws0, ws1):
    wsem = (ws0, ws1)
    wid = lax.axis_index("s") * _NC + lax.axis_index("c")
    rowbase = wid * _TB
    lanes = lax.iota(jnp.int32, 16)
    pltpu.sync_copy(wp_hbm, wp_v)
    pltpu.sync_copy(x_hbm.at[pl.ds(rowbase, _TB)], xi)
    pltpu.sync_copy(p_hbm.at[pl.ds(rowbase, _TB)], pi)

    def build_xt(ll, s):
      col = lax.broadcast(ll, (16,))
      for g in range(8):
        rows = lanes + (16 * g)
        xt[s, pl.ds(16 * g, 16)] = plsc.load_gather(xi, [rows, col])

    def issue_gather(s):
      pltpu.async_copy(ww_hbm.at[xt.at[s]], wrows, gsem)

    def wait_gather():
      pltpu.make_async_copy(ww_hbm.at[xt.at[0]], wrows, gsem).wait()

    def fill_cat(ll, cb):
      @pl.loop(0, _DT - 2)
      def _wtile(dt):
        for d8 in range(8):
          d = lax.broadcast(dt * 8 + d8, (16,))
          for g in range(8):
            cat[cb, dt, d8, pl.ds(16 * g, 16)] = plsc.load_gather(
                wrows, [lanes + (16 * g), d])

      col = lax.broadcast(ll, (16,))
      for g in range(8):
        pv = plsc.load_gather(pi, [lanes + (16 * g), col])
        for dp in range(_DP):
          cat[cb, 16 + dp // 8, dp % 8, pl.ds(16 * g, 16)] = (
              plsc.load_gather(wp_v, [pv, lax.broadcast(dp, (16,))]))

    def issue_write(l, cb):
      pltpu.async_copy(
          cat.at[cb],
          out_hbm.at[l, pl.ds(0, _DT), pl.ds(wid * 8, 8), pl.ds(0, 128)],
          wsem[cb])

    def wait_write(cb):
      pltpu.make_async_copy(
          cat.at[cb],
          out_hbm.at[0, pl.ds(0, _DT), pl.ds(wid * 8, 8), pl.ds(0, 128)],
          wsem[cb]).wait()

    build_xt(0, 0)
    issue_gather(0)
    build_xt(1, 1)
    wait_gather()
    fill_cat(0, 0)
    issue_gather(1)
    issue_write(0, 0)
    build_xt(2, 0)
    wait_gather()
    fill_cat(1, 1)
    issue_gather(0)
    issue_write(1, 1)
    wait_write(0)

    @pl.loop(2, _L - 2, step=2)
    def _tok(ll0):
      for cb in range(2):
        ll = ll0 + cb
        build_xt(ll + 1, 1 - cb)
        wait_gather()
        fill_cat(ll, cb)
        issue_gather(1 - cb)
        issue_write(ll, cb)
        wait_write(1 - cb)

    build_xt(_L - 1, 1)
    wait_gather()
    fill_cat(_L - 2, 0)
    issue_gather(1)
    issue_write(_L - 2, 0)
    wait_write(1)
    wait_gather()
    fill_cat(_L - 1, 1)
    issue_write(_L - 1, 1)
    wait_write(0)
    wait_write(1)

  return emb


_emb = _build()


@jax.jit
def kernel(x, pos, W_word, W_pos):
  out5 = _emb(x.astype(jnp.int32), pos.astype(jnp.int32),
              W_word, W_pos).reshape(_L, _DT, _NW, 8, 128)
  return out5.transpose(2, 4, 0, 1, 3).reshape(_B, _L, _D)


# speedup vs baseline: 2.2336x; 2.2336x over previous
import functools

import jax
import jax.numpy as jnp
from jax import lax
from jax.experimental import pallas as pl
from jax.experimental.pallas import tpu as pltpu
from jax.experimental.pallas import tpu_sc as plsc

_B, _L = 4096, 200
_N = _B * _L
_DW, _DP = 128, 16
_D = _DW + _DP
_NC, _NS = 2, 16
_NW = _NC * _NS
_RW = _B // _NW
_CR = 32
_NCH = _RW // _CR
_GA, _GB = 128, _L - 128


def _build():
  mesh = plsc.VectorSubcoreMesh(core_axis_name="c", subcore_axis_name="s")

  @functools.partial(
      pl.kernel,
      mesh=mesh,
      compiler_params=pltpu.CompilerParams(needs_layout_passes=False),
      out_type=jax.ShapeDtypeStruct((_B, _L, _D), jnp.float32),
      scratch_types=[
          pltpu.VMEM((_CR, _L), jnp.int32),
          pltpu.VMEM((_CR, _L), jnp.int32),
          pltpu.VMEM((24, _DP), jnp.float32),
          pltpu.VMEM((2, _L, _DW), jnp.float32),
          pltpu.VMEM((2, _L, _DP), jnp.float32),
          pltpu.SemaphoreType.DMA,
          pltpu.SemaphoreType.DMA,
          pltpu.SemaphoreType.DMA,
          pltpu.SemaphoreType.DMA,
      ],
  )
  def emb(x_hbm, p_hbm, ww_hbm, wp_hbm, out_hbm,
          xi, pi, wp_v, wrows, prows, gs0, gs1, ws0, ws1):
    gsem = (gs0, gs1)
    wsem = (ws0, ws1)
    wid = lax.axis_index("s") * _NC + lax.axis_index("c")
    rowbase = wid * _RW
    lanes = lax.iota(jnp.int32, 16)
    pltpu.sync_copy(wp_hbm, wp_v)

    def issue_gather(rl, b):
      pltpu.async_copy(ww_hbm.at[xi.at[rl, pl.ds(0, _GA)]],
                       wrows.at[b, pl.ds(0, _GA)], gsem[b])
      pltpu.async_copy(ww_hbm.at[xi.at[rl, pl.ds(_GA, _GB)]],
                       wrows.at[b, pl.ds(_GA, _GB)], gsem[b])

    def wait_gather(b):
      pltpu.make_async_copy(
          ww_hbm.at[xi.at[0, pl.ds(0, _GA)]], wrows.at[b, pl.ds(0, _GA)],
          gsem[b]).wait()
      pltpu.make_async_copy(
          ww_hbm.at[xi.at[0, pl.ds(_GA, _GB)]], wrows.at[b, pl.ds(_GA, _GB)],
          gsem[b]).wait()

    def pos_fill(rl, b):
      @pl.loop(0, _L - 8, step=16)
      def _tok(i0):
        pvec = pi[rl, pl.ds(i0, 16)]
        for j in range(16):
          row = lax.broadcast(pvec[j], (16,))
          prows[b, i0 + j, :] = plsc.load_gather(wp_v, [row, lanes])

      pvec = pi[rl, pl.ds(_L - 16, 16)]
      for j in range(8, 16):
        row = lax.broadcast(pvec[j], (16,))
        prows[b, _L - 16 + j, :] = plsc.load_gather(wp_v, [row, lanes])

    def issue_write(row, b):
      pltpu.async_copy(
          wrows.at[b], out_hbm.at[row, pl.ds(0, _L), pl.ds(0, _DW)], wsem[b])
      pltpu.async_copy(
          prows.at[b], out_hbm.at[row, pl.ds(0, _L), pl.ds(_DW, _DP)],
          wsem[b])

    def wait_write(b):
      pltpu.make_async_copy(
          wrows.at[b], out_hbm.at[0, pl.ds(0, _L), pl.ds(0, _DW)],
          wsem[b]).wait()
      pltpu.make_async_copy(
          prows.at[b], out_hbm.at[0, pl.ds(0, _L), pl.ds(_DW, _DP)],
          wsem[b]).wait()

    @pl.loop(0, _NCH)
    def _chunk(c):
      crow = rowbase + c * _CR
      pltpu.sync_copy(x_hbm.at[pl.ds(crow, _CR)], xi)
      pltpu.sync_copy(p_hbm.at[pl.ds(crow, _CR)], pi)
      issue_gather(0, 0)
      issue_gather(1, 1)

      @pl.loop(0, _CR, step=2)
      def _rows(rl0):
        for b in range(2):
          rl = rl0 + b
          pos_fill(rl, b)
          wait_gather(b)
          issue_write(crow + rl, b)
          wait_write(b)

          @pl.when(rl + 2 < _CR)
          def _():
            issue_gather(rl + 2, b)

  return emb


_emb = _build()


@jax.jit
def kernel(x, pos, W_word, W_pos):
  return _emb(x.astype(jnp.int32), pos.astype(jnp.int32), W_word, W_pos)
